# baseline (speedup 1.0000x reference)
import jax
import jax.numpy as jnp
from jax.experimental import pallas as pl
from jax.experimental.pallas import tpu as pltpu


def _make_body(batch, rows):
    def body(in_ref, out_hbm, sems):
        i = pl.program_id(0)
        copies = [
            pltpu.make_async_copy(
                in_ref, out_hbm.at[b, pl.ds(i * rows, rows), :], sems.at[b]
            )
            for b in range(batch)
        ]
        for c in copies:
            c.start()
        for c in copies:
            c.wait()

    return body


def kernel(x, pe_weight):
    batch = x.shape[0]
    max_len, d = pe_weight.shape
    rows = 512
    return pl.pallas_call(
        _make_body(batch, rows),
        grid=(max_len // rows,),
        in_specs=[pl.BlockSpec((rows, d), lambda i: (i, 0))],
        out_specs=pl.BlockSpec(memory_space=pl.ANY),
        out_shape=jax.ShapeDtypeStruct((batch, max_len, d), pe_weight.dtype),
        scratch_shapes=[pltpu.SemaphoreType.DMA((batch,))],
    )(pe_weight)


# speedup vs baseline: 1.1183x; 1.1183x over previous
import jax
import jax.numpy as jnp
from jax.experimental import pallas as pl
from jax.experimental.pallas import tpu as pltpu


def _make_body(batch):
    def body(in_ref, out_hbm, sems):
        copies = [
            pltpu.make_async_copy(in_ref, out_hbm.at[b], sems.at[b])
            for b in range(batch)
        ]
        for c in copies:
            c.start()
        for c in copies:
            c.wait()

    return body


def kernel(x, pe_weight):
    batch = x.shape[0]
    max_len, d = pe_weight.shape
    return pl.pallas_call(
        _make_body(batch),
        grid=(1,),
        in_specs=[pl.BlockSpec((max_len, d), lambda i: (0, 0))],
        out_specs=pl.BlockSpec(memory_space=pl.ANY),
        out_shape=jax.ShapeDtypeStruct((batch, max_len, d), pe_weight.dtype),
        scratch_shapes=[pltpu.SemaphoreType.DMA((batch,))],
    )(pe_weight)


# speedup vs baseline: 1.1284x; 1.0090x over previous
import jax
import jax.numpy as jnp
from jax.experimental import pallas as pl
from jax.experimental.pallas import tpu as pltpu


def _make_body(batch, max_len, d, chunk):
    nchunks = max_len // chunk

    def body(in_hbm, out_hbm, buf, in_sems, out_sems):
        def read(i):
            return pltpu.make_async_copy(
                in_hbm.at[pl.ds(i * chunk, chunk), :], buf.at[i % 2], in_sems.at[i % 2]
            )

        def write(i, b):
            return pltpu.make_async_copy(
                buf.at[i % 2],
                out_hbm.at[b, pl.ds(i * chunk, chunk), :],
                out_sems.at[i % 2, b],
            )

        read(0).start()
        for i in range(nchunks):
            if i + 1 < nchunks:
                if i >= 1:
                    for b in range(batch):
                        write(i - 1, b).wait()
                read(i + 1).start()
            read(i).wait()
            for b in range(batch):
                write(i, b).start()
        for i in (nchunks - 2, nchunks - 1):
            for b in range(batch):
                write(i, b).wait()

    return body


def kernel(x, pe_weight):
    batch = x.shape[0]
    max_len, d = pe_weight.shape
    chunk = 1024
    return pl.pallas_call(
        _make_body(batch, max_len, d, chunk),
        grid=(1,),
        in_specs=[pl.BlockSpec(memory_space=pl.ANY)],
        out_specs=pl.BlockSpec(memory_space=pl.ANY),
        out_shape=jax.ShapeDtypeStruct((batch, max_len, d), pe_weight.dtype),
        scratch_shapes=[
            pltpu.VMEM((2, chunk, d), pe_weight.dtype),
            pltpu.SemaphoreType.DMA((2,)),
            pltpu.SemaphoreType.DMA((2, batch)),
        ],
    )(pe_weight)
